# baseline (speedup 1.0000x reference)
import functools

import jax
import jax.numpy as jnp
from jax.experimental import pallas as pl
from jax.experimental.pallas import tpu as pltpu

_S0 = 104
_S1 = 296
_BM = 400


def _body(adj, xh, w_ref, b_ref, out, buf0, buf1, bufb, xv, ob0, ob1, obb,
          csem0, csem1, csemb, ssem0, ssem1, ssemb, xsem, *, nbig):
    n = xv.shape[0]
    xcp = pltpu.make_async_copy(xh, xv, xsem)
    cp0 = pltpu.make_async_copy(adj.at[pl.ds(0, _S0), :], buf0, csem0)
    cp1 = pltpu.make_async_copy(adj.at[pl.ds(_S0, _S1), :], buf1, csem1)

    def big_in(j, slot):
        return pltpu.make_async_copy(
            adj.at[pl.ds(_S0 + _S1 + j * _BM, _BM), :],
            bufb.at[slot], csemb.at[slot])

    def big_out(j, slot):
        return pltpu.make_async_copy(
            obb.at[slot], out.at[pl.ds(_S0 + _S1 + j * _BM, _BM), :],
            ssemb.at[slot])

    xcp.start()
    cp0.start()
    cp1.start()
    big_in(0, 0).start()
    big_in(1, 1).start()

    def fold(a):
        ax = jnp.dot(a, xv[...], preferred_element_type=jnp.float32)
        return (
            jnp.dot(ax, w_ref[...], preferred_element_type=jnp.float32)
            + b_ref[...]
        )

    xcp.wait()
    cp0.wait()
    ob0[...] = fold(buf0[...])
    pltpu.make_async_copy(ob0, out.at[pl.ds(0, _S0), :], ssem0).start()
    cp1.wait()
    ob1[...] = fold(buf1[...])
    pltpu.make_async_copy(ob1, out.at[pl.ds(_S0, _S1), :], ssem1).start()

    def loop(j, carry):
        slot = jax.lax.rem(j, 2)
        big_in(j, slot).wait()
        res = fold(bufb[slot])

        @pl.when(j >= 2)
        def _():
            big_out(j - 2, slot).wait()

        obb[slot] = res
        big_out(j, slot).start()

        @pl.when(j < nbig - 2)
        def _():
            big_in(j + 2, slot).start()

        return carry

    jax.lax.fori_loop(0, nbig, loop, 0)

    pltpu.make_async_copy(ob0, out.at[pl.ds(0, _S0), :], ssem0).wait()
    pltpu.make_async_copy(ob1, out.at[pl.ds(_S0, _S1), :], ssem1).wait()
    big_out(nbig - 2, jax.lax.rem(nbig - 2, 2)).wait()
    big_out(nbig - 1, jax.lax.rem(nbig - 1, 2)).wait()


def kernel(x, adj, W, b):
    n, din = x.shape
    dout = W.shape[1]
    b2 = b.reshape(1, dout)
    nbig = (n - _S0 - _S1) // _BM
    return pl.pallas_call(
        functools.partial(_body, nbig=nbig),
        in_specs=[
            pl.BlockSpec(memory_space=pltpu.MemorySpace.HBM),
            pl.BlockSpec(memory_space=pltpu.MemorySpace.HBM),
            pl.BlockSpec(memory_space=pltpu.MemorySpace.VMEM),
            pl.BlockSpec(memory_space=pltpu.MemorySpace.VMEM),
        ],
        out_specs=pl.BlockSpec(memory_space=pltpu.MemorySpace.HBM),
        out_shape=jax.ShapeDtypeStruct((n, dout), jnp.float32),
        scratch_shapes=[
            pltpu.VMEM((_S0, n), jnp.float32),
            pltpu.VMEM((_S1, n), jnp.float32),
            pltpu.VMEM((2, _BM, n), jnp.float32),
            pltpu.VMEM((n, din), jnp.float32),
            pltpu.VMEM((_S0, dout), jnp.float32),
            pltpu.VMEM((_S1, dout), jnp.float32),
            pltpu.VMEM((2, _BM, dout), jnp.float32),
            pltpu.SemaphoreType.DMA,
            pltpu.SemaphoreType.DMA,
            pltpu.SemaphoreType.DMA((2,)),
            pltpu.SemaphoreType.DMA,
            pltpu.SemaphoreType.DMA,
            pltpu.SemaphoreType.DMA((2,)),
            pltpu.SemaphoreType.DMA,
        ],
    )(adj, x, W, b2)


# speedup vs baseline: 1.0120x; 1.0120x over previous
import jax
import jax.numpy as jnp
from jax.experimental import pallas as pl
from jax.experimental.pallas import tpu as pltpu

_BM = 400


def _gcn_body(adj_ref, x_ref, w_ref, b_ref, out_ref):
    ax = jnp.dot(
        adj_ref[...],
        x_ref[...],
        preferred_element_type=jnp.float32,
        precision=jax.lax.Precision.DEFAULT,
    )
    out_ref[...] = (
        jnp.dot(ax, w_ref[...], preferred_element_type=jnp.float32) + b_ref[...]
    )


def kernel(x, adj, W, b):
    n, din = x.shape
    dout = W.shape[1]
    b2 = b.reshape(1, dout)
    return pl.pallas_call(
        _gcn_body,
        grid=(pl.cdiv(n, _BM),),
        in_specs=[
            pl.BlockSpec((_BM, n), lambda i: (i, 0)),
            pl.BlockSpec((n, din), lambda i: (0, 0)),
            pl.BlockSpec((din, dout), lambda i: (0, 0)),
            pl.BlockSpec((1, dout), lambda i: (0, 0)),
        ],
        out_specs=pl.BlockSpec((_BM, dout), lambda i: (i, 0)),
        out_shape=jax.ShapeDtypeStruct((n, dout), jnp.float32),
        compiler_params=pltpu.CompilerParams(
            dimension_semantics=("parallel",),
        ),
    )(adj, x, W, b2)
